# baseline (speedup 1.0000x reference)
import jax
import jax.numpy as jnp
from jax.experimental import pallas as pl

MAX_LEN_ = 200
D_MODEL_ = 64
SB_ = 8
LB_ = 1024


def _bcast_body(pe_ref, out_ref):
    out_ref[...] = jnp.broadcast_to(pe_ref[...][..., None], out_ref.shape)


def kernel(x, pe_weight):
    batch = x.shape[0]
    out_p = pl.pallas_call(
        _bcast_body,
        grid=(MAX_LEN_ // SB_, batch // LB_),
        in_specs=[pl.BlockSpec((SB_, D_MODEL_), lambda i, j: (i, 0))],
        out_specs=pl.BlockSpec((SB_, D_MODEL_, LB_), lambda i, j: (i, 0, j)),
        out_shape=jax.ShapeDtypeStruct((MAX_LEN_, D_MODEL_, batch), pe_weight.dtype),
    )(pe_weight)
    return jnp.transpose(out_p, (2, 0, 1))


# speedup vs baseline: 1.2696x; 1.2696x over previous
import jax
import jax.numpy as jnp
from jax.experimental import pallas as pl

MAX_LEN_ = 200
D_MODEL_ = 64
SB_ = 8


def _bcast_body(pe_ref, out_ref):
    out_ref[...] = jnp.broadcast_to(pe_ref[...][..., None], out_ref.shape)


def kernel(x, pe_weight):
    batch = x.shape[0]
    out_p = pl.pallas_call(
        _bcast_body,
        grid=(MAX_LEN_ // SB_,),
        in_specs=[pl.BlockSpec((SB_, D_MODEL_), lambda i: (i, 0))],
        out_specs=pl.BlockSpec((SB_, D_MODEL_, batch), lambda i: (i, 0, 0)),
        out_shape=jax.ShapeDtypeStruct((MAX_LEN_, D_MODEL_, batch), pe_weight.dtype),
    )(pe_weight)
    return jnp.transpose(out_p, (2, 0, 1))
